# Initial kernel scaffold; baseline (speedup 1.0000x reference)
import jax
import jax.numpy as jnp
from jax.experimental import pallas as pl


def kernel(x, edge_index, Wl1, Wr1, b1, Wl2, Wr2, b2):
    raise NotImplementedError("write your pallas kernel here")


# speedup vs baseline: 7.3336x; 7.3336x over previous
import functools

import jax
import jax.numpy as jnp
from jax import lax
from jax.experimental import pallas as pl
from jax.experimental.pallas import tpu as pltpu
from jax.experimental.pallas import tpu_sc as plsc

NC = 2
NS = 16
NW = NC * NS
L = 16
CHUNK = 128


@functools.lru_cache(maxsize=None)
def _make_seg_sum(n_pad: int, width: int, k: int):
    mesh = plsc.VectorSubcoreMesh(core_axis_name="c", subcore_axis_name="s")
    rows_per_sub = n_pad // NS
    zchunks = rows_per_sub // CHUNK

    @functools.partial(
        pl.kernel,
        out_type=jax.ShapeDtypeStruct((NC, n_pad, width), jnp.float32),
        mesh=mesh,
        scratch_types=[
            pltpu.VMEM((k, CHUNK), jnp.int32),
            pltpu.VMEM((k, CHUNK), jnp.int32),
            pltpu.VMEM((CHUNK, width), jnp.float32),
            pltpu.VMEM_SHARED((n_pad, width), jnp.float32),
            pltpu.SemaphoreType.DMA,
        ],
        compiler_params=pltpu.CompilerParams(use_tc_tiling_on_sc=False),
    )
    def seg(tbl_hbm, src_hbm, dst_hbm, out_hbm, src_v, dst_v, rows_v, acc, sem):
        cid = lax.axis_index("c")
        sid = lax.axis_index("s")
        pltpu.sync_copy(src_hbm.at[cid, sid], src_v)
        pltpu.sync_copy(dst_hbm.at[cid, sid], dst_v)

        zero = jnp.zeros((L,), jnp.float32)

        @pl.loop(0, CHUNK)
        def _(i):
            for c in range(width // L):
                rows_v[i, pl.ds(c * L, L)] = zero

        base = sid * rows_per_sub

        @pl.loop(0, zchunks)
        def _(z):
            pltpu.sync_copy(rows_v, acc.at[pl.ds(base + z * CHUNK, CHUNK)])

        plsc.subcore_barrier()

        @pl.loop(0, k)
        def _(j):
            pltpu.async_copy(tbl_hbm.at[src_v.at[j]], rows_v, sem).wait()
            pltpu.sync_copy(rows_v, acc.at[dst_v.at[j]], add=True)

        plsc.subcore_barrier()

        @pl.loop(0, zchunks)
        def _(z):
            sl = pl.ds(base + z * CHUNK, CHUNK)
            pltpu.sync_copy(acc.at[sl], rows_v)
            pltpu.sync_copy(rows_v, out_hbm.at[cid, sl])

    return seg


_DN = (((1,), (1,)), ((), ()))


def _proj1(x, Wl1, Wr1, bn):
    n, d = x.shape
    h = Wl1.shape[0]
    w1 = h + L

    def body(x_ref, wl_ref, wr_ref, aug_ref, xr_ref):
        xb = x_ref[...]
        xl = lax.dot_general(xb, wl_ref[...], _DN,
                             preferred_element_type=jnp.float32)
        tail = (lax.broadcasted_iota(jnp.int32, (bn, L), 1) == 0)
        aug_ref[...] = jnp.concatenate([xl, tail.astype(jnp.float32)], axis=1)
        xr_ref[...] = lax.dot_general(xb, wr_ref[...], _DN,
                                      preferred_element_type=jnp.float32)

    return pl.pallas_call(
        body,
        grid=(n // bn,),
        in_specs=[
            pl.BlockSpec((bn, d), lambda i: (i, 0)),
            pl.BlockSpec((h, d), lambda i: (0, 0)),
            pl.BlockSpec((h, d), lambda i: (0, 0)),
        ],
        out_specs=[
            pl.BlockSpec((bn, w1), lambda i: (i, 0)),
            pl.BlockSpec((bn, h), lambda i: (i, 0)),
        ],
        out_shape=[
            jax.ShapeDtypeStruct((n, w1), jnp.float32),
            jax.ShapeDtypeStruct((n, h), jnp.float32),
        ],
    )(x, Wl1, Wr1)


def _mid(acc1, xr, b1, W16, bb, bn):
    n, h = xr.shape
    n_pad = acc1.shape[1]
    w1 = acc1.shape[2]

    def body(acc_ref, xr_ref, b1_ref, w16_ref, bb_ref, tbl_ref):
        p = acc_ref[0] + acc_ref[1]
        deg = jnp.maximum(p[:, h:h + 1], 1.0)
        hid = jnp.maximum(p[:, :h] / deg + b1_ref[...] + xr_ref[...], 0.0)
        t = lax.dot_general(hid, w16_ref[...], _DN,
                            preferred_element_type=jnp.float32)
        col = lax.broadcasted_iota(jnp.int32, (bn, L), 1)
        tbl_ref[...] = t + bb_ref[...] + deg * (col == 4).astype(jnp.float32)

    return pl.pallas_call(
        body,
        grid=(n // bn,),
        in_specs=[
            pl.BlockSpec((NC, bn, w1), lambda i: (0, i, 0)),
            pl.BlockSpec((bn, h), lambda i: (i, 0)),
            pl.BlockSpec((1, h), lambda i: (0, 0)),
            pl.BlockSpec((L, h), lambda i: (0, 0)),
            pl.BlockSpec((1, L), lambda i: (0, 0)),
        ],
        out_specs=pl.BlockSpec((bn, L), lambda i: (i, 0)),
        out_shape=jax.ShapeDtypeStruct((n, L), jnp.float32),
    )(acc1, xr, b1, W16, bb)


def _final(acc2, tbl2, o, bn):
    n = tbl2.shape[0]

    def body(acc_ref, tbl_ref, out_ref):
        s = acc_ref[0] + acc_ref[1]
        deg = tbl_ref[:, 4:5]
        y = s[:, :o] / deg + tbl_ref[:, 8:8 + o]
        m = jnp.max(y, axis=1, keepdims=True)
        e = jnp.exp(y - m)
        out_ref[...] = e / jnp.sum(e, axis=1, keepdims=True)

    return pl.pallas_call(
        body,
        grid=(n // bn,),
        in_specs=[
            pl.BlockSpec((NC, bn, L), lambda i: (0, i, 0)),
            pl.BlockSpec((bn, L), lambda i: (i, 0)),
        ],
        out_specs=pl.BlockSpec((bn, o), lambda i: (i, 0)),
        out_shape=jax.ShapeDtypeStruct((n, o), jnp.float32),
    )(acc2, tbl2)


def kernel(x, edge_index, Wl1, Wr1, b1, Wl2, Wr2, b2):
    n, d = x.shape
    e = edge_index.shape[1]
    h = Wl1.shape[0]
    o = Wl2.shape[0]

    k = -(-e // (NW * CHUNK))
    e_pad = k * NW * CHUNK
    n_pad = -(-(n + 1) // (NS * CHUNK)) * (NS * CHUNK)
    bn = 1000 if n % 1000 == 0 else 8

    src = edge_index[0]
    dst = edge_index[1]
    pad = e_pad - e
    srcp = jnp.concatenate([src, jnp.zeros((pad,), jnp.int32)])
    dstp = jnp.concatenate([dst, jnp.full((pad,), n, jnp.int32)])
    srcp = srcp.reshape(NC, NS, k, CHUNK)
    dstp = dstp.reshape(NC, NS, k, CHUNK)

    aug, xr = _proj1(x, Wl1, Wr1, bn)
    acc1 = _make_seg_sum(n_pad, h + L, k)(aug, srcp, dstp)

    W16 = jnp.zeros((L, h), jnp.float32).at[0:o].set(Wl2).at[8:8 + o].set(Wr2)
    bb = jnp.zeros((1, L), jnp.float32).at[0, 8:8 + o].set(b2)
    tbl2 = _mid(acc1, xr, b1.reshape(1, h), W16, bb, bn)

    acc2 = _make_seg_sum(n_pad, L, k)(tbl2, srcp, dstp)
    return _final(acc2, tbl2, o, bn)
